# Initial kernel scaffold; baseline (speedup 1.0000x reference)
import jax
import jax.numpy as jnp
from jax.experimental import pallas as pl


def kernel(x, edge_index, edge_attr, W1, att_src1, att_dst1, We1, att_e1, b1, W2, att_src2, att_dst2, We2, att_e2, b2, A1, ab1, A2, ab2, M1, mb1, M2, mb2, M3, mb3):
    raise NotImplementedError("write your pallas kernel here")


# speedup vs baseline: 19.5201x; 19.5201x over previous
import functools

import jax
import jax.numpy as jnp
from jax import lax
from jax.experimental import pallas as pl
from jax.experimental.pallas import tpu as pltpu
from jax.experimental.pallas import tpu_sc as plsc

_N = 10000
_E = 320000
_H = 128
_EDIM = 16

_NTILES = 32
_EPT = _E // _NTILES
_CH = 80
_NCH = _EPT // _CH
_ROW_CHUNK = 624
_LAST_ROWS = _N - 15 * _ROW_CHUNK

_f32 = jnp.float32
_i32 = jnp.int32


def _lrelu(x, slope):
    return jnp.maximum(x, x * slope)


def _tc_prep_body(x_ref, w_ref, asrc_ref, adst_ref, h_ref, as_ref, ad_ref):
    h = jnp.dot(x_ref[...], w_ref[...], preferred_element_type=_f32)
    h_ref[...] = h
    as_ref[...] = jnp.dot(h, asrc_ref[...], preferred_element_type=_f32)
    ad_ref[...] = jnp.dot(h, adst_ref[...], preferred_element_type=_f32)


def _tc_prep(x, w, asrc, adst):
    return pl.pallas_call(
        _tc_prep_body,
        out_shape=[
            jax.ShapeDtypeStruct((_N, _H), _f32),
            jax.ShapeDtypeStruct((_N, 1), _f32),
            jax.ShapeDtypeStruct((_N, 1), _f32),
        ],
    )(x, w, asrc, adst)


def _tc_edge_body(x_ref, b1_ref, b2_ref, ae1_ref, ae2_ref, m1_ref, m2_ref):
    a1 = jnp.dot(x_ref[...], b1_ref[...], preferred_element_type=_f32)
    a2 = jnp.dot(x_ref[...], b2_ref[...], preferred_element_type=_f32)
    ae1_ref[...] = a1
    ae2_ref[...] = a2
    m1_ref[...] = jnp.sum(a1).reshape(1, 1) * (1.0 / _E)
    m2_ref[...] = jnp.sum(a2).reshape(1, 1) * (1.0 / _E)


def _tc_edge(x_fold, b1, b2):
    r = _E // _H
    return pl.pallas_call(
        _tc_edge_body,
        out_shape=[
            jax.ShapeDtypeStruct((r, _H), _f32),
            jax.ShapeDtypeStruct((r, _H), _f32),
            jax.ShapeDtypeStruct((1, 1), _f32),
            jax.ShapeDtypeStruct((1, 1), _f32),
        ],
    )(x_fold, b1, b2)


def _sc_layer_body(src_hbm, dst_hbm, as_hbm, ad_hbm, ae_hbm, h_hbm,
                   numer_out, denom_out,
                   as_v, ad_v, src_cur, dst_cur, ae_cur, ex_cur,
                   rows_v, zrows,
                   numer_sp, denom_sp):
    cid = lax.axis_index("c")
    sid = lax.axis_index("s")
    wid = cid * 16 + sid
    base = wid * _EPT

    pltpu.sync_copy(as_hbm, as_v)
    pltpu.sync_copy(ad_hbm, ad_v)

    @pl.loop(0, 16)
    def _zz(j):
        for c in range(8):
            zrows[j, pl.ds(c * 16, 16)] = jnp.zeros((16,), _f32)

    row0 = sid * _ROW_CHUNK
    for r in range(0, _LAST_ROWS, 16):
        pltpu.sync_copy(zrows.at[pl.ds(0, 16), :],
                        numer_sp.at[pl.ds(row0 + r, 16), :])

    @pl.when(sid == 0)
    def _():
        for r in range(0, _N, 128):
            n = min(128, _N - r)
            pltpu.sync_copy(zrows.at[0, pl.ds(0, n)],
                            denom_sp.at[pl.ds(r, n)])

    plsc.subcore_barrier()

    @pl.loop(0, _NCH)
    def _vec(g):
        off = base + g * _CH
        pltpu.sync_copy(src_hbm.at[pl.ds(off, _CH)], src_cur)
        pltpu.sync_copy(dst_hbm.at[pl.ds(off, _CH)], dst_cur)
        pltpu.sync_copy(ae_hbm.at[pl.ds(off, _CH)], ae_cur)
        pltpu.sync_copy(h_hbm.at[src_cur], rows_v)

        @pl.loop(0, _CH // 16)
        def _scal(q):
            qo = q * 16
            a = plsc.load_gather(as_v, [src_cur[pl.ds(qo, 16)]])
            b = plsc.load_gather(ad_v, [dst_cur[pl.ds(qo, 16)]])
            alpha = a + b + ae_cur[pl.ds(qo, 16)]
            ex_cur[pl.ds(qo, 16)] = jnp.exp(_lrelu(alpha, 0.2))

        @pl.loop(0, _CH // 16)
        def _mul(q):
            exv = ex_cur[pl.ds(q * 16, 16)]
            for jj in range(16):
                j = q * 16 + jj
                s = exv[jj]
                for c in range(8):
                    rows_v[j, pl.ds(c * 16, 16)] = (
                        rows_v[j, pl.ds(c * 16, 16)] * s)

        pltpu.sync_copy(rows_v, numer_sp.at[dst_cur], add=True)
        pltpu.sync_copy(ex_cur, denom_sp.at[dst_cur], add=True)

    plsc.subcore_barrier()

    @pl.when(sid < 15)
    def _():
        pltpu.sync_copy(numer_sp.at[pl.ds(row0, _ROW_CHUNK), :],
                        numer_out.at[cid, pl.ds(row0, _ROW_CHUNK), :])

    @pl.when(sid == 15)
    def _():
        pltpu.sync_copy(numer_sp.at[pl.ds(15 * _ROW_CHUNK, _LAST_ROWS), :],
                        numer_out.at[cid, pl.ds(15 * _ROW_CHUNK, _LAST_ROWS), :])

    @pl.when(sid == 0)
    def _():
        pltpu.sync_copy(denom_sp, denom_out.at[cid, 0])


def _sc_layer(src, dst, asv, adv, ae, h):
    mesh = plsc.VectorSubcoreMesh(core_axis_name="c", subcore_axis_name="s",
                                  num_cores=2, num_subcores=16)
    f = pl.kernel(
        _sc_layer_body,
        out_type=[
            jax.ShapeDtypeStruct((2, _N, _H), _f32),
            jax.ShapeDtypeStruct((2, 1, _N), _f32),
        ],
        mesh=mesh,
        compiler_params=pltpu.CompilerParams(needs_layout_passes=False),
        scratch_types=[
            pltpu.VMEM((_N,), _f32),
            pltpu.VMEM((_N,), _f32),
            pltpu.VMEM((_CH,), _i32),
            pltpu.VMEM((_CH,), _i32),
            pltpu.VMEM((_CH,), _f32),
            pltpu.VMEM((_CH,), _f32),
            pltpu.VMEM((_CH, _H), _f32),
            pltpu.VMEM((16, _H), _f32),
            pltpu.VMEM_SHARED((_N, _H), _f32),
            pltpu.VMEM_SHARED((_N,), _f32),
        ],
    )
    return f(src, dst, asv, adv, ae, h)


def _tc_mid_body(n0, n1, d0, d1, h1, as1, ad1, m1, b1, w2, s2, t2,
                 h2_ref, as2_ref, ad2_ref):
    exl = jnp.exp(_lrelu(as1[...] + ad1[...] + m1[...], 0.2))
    num = n0[...] + n1[...] + exl * h1[...]
    den = d0[...] + d1[...] + exl + 1e-16
    out = num / den + b1[...]
    hr = jnp.maximum(out, 0.0)
    h2 = jnp.dot(hr, w2[...], preferred_element_type=_f32)
    h2_ref[...] = h2
    as2_ref[...] = jnp.dot(h2, s2[...], preferred_element_type=_f32)
    ad2_ref[...] = jnp.dot(h2, t2[...], preferred_element_type=_f32)


def _tc_mid(n0, n1, d0, d1, h1, as1, ad1, m1, b1, w2, s2, t2):
    return pl.pallas_call(
        _tc_mid_body,
        out_shape=[
            jax.ShapeDtypeStruct((_N, _H), _f32),
            jax.ShapeDtypeStruct((_N, 1), _f32),
            jax.ShapeDtypeStruct((_N, 1), _f32),
        ],
    )(n0, n1, d0, d1, h1, as1, ad1, m1, b1, w2, s2, t2)


def _tc_head_body(n0, n1, d0, d1, h2, as2, ad2, m2, b2,
                  a1w, ab1, a2w, ab2, m1w, mb1, m2w, mb2, m3w, mb3,
                  ap_ref, ns_ref):
    exl = jnp.exp(_lrelu(as2[...] + ad2[...] + m2[...], 0.2))
    num = n0[...] + n1[...] + exl * h2[...]
    den = d0[...] + d1[...] + exl + 1e-16
    h = num / den + b2[...]

    emb = jnp.mean(h, axis=0, keepdims=True)
    a = _lrelu(jnp.dot(emb, a1w[...], preferred_element_type=_f32) + ab1[...], 0.01)
    a = _lrelu(jnp.dot(a, a2w[...], preferred_element_type=_f32) + ab2[...], 0.01)
    amx = jnp.max(a)
    e = jnp.exp(a - amx)
    ap_ref[...] = e / jnp.sum(e)

    z = _lrelu(jnp.dot(h, m1w[...], preferred_element_type=_f32) + mb1[...], 0.01)
    z = _lrelu(jnp.dot(z, m2w[...], preferred_element_type=_f32) + mb2[...], 0.01)
    t = jnp.dot(z, m3w[...], preferred_element_type=_f32)[:, 0:1] + mb3[...]
    ns_ref[...] = 1.0 / (1.0 + jnp.exp(-t))


def _tc_head(n0, n1, d0, d1, h2, as2, ad2, m2, b2,
             a1w, ab1, a2w, ab2, m1w, mb1, m2w, mb2, m3w, mb3):
    return pl.pallas_call(
        _tc_head_body,
        out_shape=[
            jax.ShapeDtypeStruct((1, _H), _f32),
            jax.ShapeDtypeStruct((_N, 1), _f32),
        ],
    )(n0, n1, d0, d1, h2, as2, ad2, m2, b2,
      a1w, ab1, a2w, ab2, m1w, mb1, m2w, mb2, m3w, mb3)


def kernel(x, edge_index, edge_attr, W1, att_src1, att_dst1, We1, att_e1, b1,
           W2, att_src2, att_dst2, We2, att_e2, b2, A1, ab1, A2, ab2,
           M1, mb1, M2, mb2, M3, mb3):
    asrc1 = att_src1[:, None]
    adst1 = att_dst1[:, None]
    asrc2 = att_src2[:, None]
    adst2 = att_dst2[:, None]
    ate1 = att_e1[:, None]
    ate2 = att_e2[:, None]
    b1r = b1[None, :]
    b2r = b2[None, :]
    ab1r = ab1[None, :]
    mb1r = mb1[None, :]
    mb2r = mb2[None, :]
    mb3r = mb3[None, :]
    a2p = jnp.zeros((_H, _H), _f32).at[:, :3].set(A2)
    ab2p = jnp.full((1, _H), -1e30, _f32).at[0, :3].set(ab2)
    m3p = jnp.zeros((_H, _H), _f32).at[:, 0:1].set(M3)

    v1 = We1 @ att_e1
    v2 = We2 @ att_e2
    rows_idx = (_EDIM * jnp.arange(_H)[:, None]
                + jnp.arange(_EDIM)[None, :]).reshape(-1)
    cols_idx = jnp.repeat(jnp.arange(_H), _EDIM)
    b1f = jnp.zeros((_H * _EDIM, _H), _f32).at[rows_idx, cols_idx].set(
        jnp.tile(v1, _H))
    b2f = jnp.zeros((_H * _EDIM, _H), _f32).at[rows_idx, cols_idx].set(
        jnp.tile(v2, _H))
    x_fold = edge_attr.reshape(_E // _H, _H * _EDIM)

    h1, as1, ad1 = _tc_prep(x, W1, asrc1, adst1)
    ae1, ae2, m1, m2 = _tc_edge(x_fold, b1f, b2f)

    src = edge_index[0]
    dst = edge_index[1]
    numer1, denom1 = _sc_layer(src, dst, as1.reshape(-1), ad1.reshape(-1),
                               ae1.reshape(-1), h1)
    h2, as2, ad2 = _tc_mid(numer1[0], numer1[1],
                           denom1[0, 0][:, None], denom1[1, 0][:, None],
                           h1, as1, ad1, m1, b1r, W2, asrc2, adst2)

    numer2, denom2 = _sc_layer(src, dst, as2.reshape(-1), ad2.reshape(-1),
                               ae2.reshape(-1), h2)
    ap, ns = _tc_head(numer2[0], numer2[1],
                      denom2[0, 0][:, None], denom2[1, 0][:, None],
                      h2, as2, ad2, m2, b2r,
                      A1, ab1r, a2p, ab2p, M1, mb1r, M2, mb2r, m3p, mb3r)

    return (ap[0, :3], ns[:, 0])


# speedup vs baseline: 31.6424x; 1.6210x over previous
import functools

import jax
import jax.numpy as jnp
from jax import lax
from jax.experimental import pallas as pl
from jax.experimental.pallas import tpu as pltpu
from jax.experimental.pallas import tpu_sc as plsc

_N = 10000
_E = 320000
_H = 128
_EDIM = 16

_NTILES = 32
_EPT = _E // _NTILES
_CH = 80
_NCH = _EPT // _CH
_ROW_CHUNK = 624
_LAST_ROWS = _N - 15 * _ROW_CHUNK

_f32 = jnp.float32
_i32 = jnp.int32


def _lrelu(x, slope):
    return jnp.maximum(x, x * slope)


def _tc_prep_body(x_ref, w_ref, asrc_ref, adst_ref, h_ref, as_ref, ad_ref):
    h = jnp.dot(x_ref[...], w_ref[...], preferred_element_type=_f32)
    h_ref[...] = h
    as_ref[...] = jnp.dot(h, asrc_ref[...], preferred_element_type=_f32)
    ad_ref[...] = jnp.dot(h, adst_ref[...], preferred_element_type=_f32)


def _tc_prep(x, w, asrc, adst):
    return pl.pallas_call(
        _tc_prep_body,
        out_shape=[
            jax.ShapeDtypeStruct((_N, _H), _f32),
            jax.ShapeDtypeStruct((_N, 1), _f32),
            jax.ShapeDtypeStruct((_N, 1), _f32),
        ],
    )(x, w, asrc, adst)


def _tc_edge_body(x_ref, b1_ref, b2_ref, ae1_ref, ae2_ref, m1_ref, m2_ref):
    a1 = jnp.dot(x_ref[...], b1_ref[...], preferred_element_type=_f32)
    a2 = jnp.dot(x_ref[...], b2_ref[...], preferred_element_type=_f32)
    ae1_ref[...] = a1
    ae2_ref[...] = a2
    m1_ref[...] = jnp.sum(a1).reshape(1, 1) * (1.0 / _E)
    m2_ref[...] = jnp.sum(a2).reshape(1, 1) * (1.0 / _E)


def _tc_edge(x_fold, b1, b2):
    r = _E // _H
    return pl.pallas_call(
        _tc_edge_body,
        out_shape=[
            jax.ShapeDtypeStruct((r, _H), _f32),
            jax.ShapeDtypeStruct((r, _H), _f32),
            jax.ShapeDtypeStruct((1, 1), _f32),
            jax.ShapeDtypeStruct((1, 1), _f32),
        ],
    )(x_fold, b1, b2)


def _sc_layer_body(src_hbm, dst_hbm, as_hbm, ad_hbm, ae_hbm, h_hbm,
                   numer_out, denom_out,
                   as_v, ad_v,
                   src_cur0, src_cur1, dst_cur0, dst_cur1,
                   ae_cur0, ae_cur1, ex_cur0, ex_cur1,
                   rows0, rows1, zrows,
                   isem0, isem1, gsem0, gsem1, ssem0, ssem1, dsem0, dsem1,
                   numer_sp, denom_sp):
    cid = lax.axis_index("c")
    sid = lax.axis_index("s")
    wid = cid * 16 + sid
    base = wid * _EPT

    pltpu.sync_copy(as_hbm, as_v)
    pltpu.sync_copy(ad_hbm, ad_v)

    @pl.loop(0, 16)
    def _zz(j):
        for c in range(8):
            zrows[j, pl.ds(c * 16, 16)] = jnp.zeros((16,), _f32)

    row0 = sid * _ROW_CHUNK
    for r in range(0, _LAST_ROWS, 16):
        pltpu.sync_copy(zrows.at[pl.ds(0, 16), :],
                        numer_sp.at[pl.ds(row0 + r, 16), :])

    @pl.when(sid == 0)
    def _():
        for r in range(0, _N, 128):
            n = min(128, _N - r)
            pltpu.sync_copy(zrows.at[0, pl.ds(0, n)],
                            denom_sp.at[pl.ds(r, n)])

    plsc.subcore_barrier()

    srcb = (src_cur0, src_cur1)
    dstb = (dst_cur0, dst_cur1)
    aeb = (ae_cur0, ae_cur1)
    exb = (ex_cur0, ex_cur1)
    rowsb = (rows0, rows1)
    isems = (isem0, isem1)
    gsems = (gsem0, gsem1)
    ssems = (ssem0, ssem1)
    dsems = (dsem0, dsem1)

    def _issue_idx(g, b):
        off = base + g * _CH
        pltpu.async_copy(src_hbm.at[pl.ds(off, _CH)], srcb[b], isems[b])
        pltpu.async_copy(dst_hbm.at[pl.ds(off, _CH)], dstb[b], isems[b])
        pltpu.async_copy(ae_hbm.at[pl.ds(off, _CH)], aeb[b], isems[b])

    def _wait_idx(g, b):
        off = base + g * _CH
        pltpu.make_async_copy(src_hbm.at[pl.ds(off, _CH)], srcb[b],
                              isems[b]).wait()
        pltpu.make_async_copy(dst_hbm.at[pl.ds(off, _CH)], dstb[b],
                              isems[b]).wait()
        pltpu.make_async_copy(ae_hbm.at[pl.ds(off, _CH)], aeb[b],
                              isems[b]).wait()

    def _wait_scatters(b):
        pltpu.make_async_copy(rowsb[b], numer_sp.at[dstb[b]], ssems[b]).wait()
        pltpu.make_async_copy(exb[b], denom_sp.at[dstb[b]], dsems[b]).wait()

    def _process(g, b, nb, last):
        if isinstance(g, int):
            if g > 0:
                _wait_scatters(nb)
        else:
            @pl.when(g > 0)
            def _():
                _wait_scatters(nb)

        if not last:
            _issue_idx(g + 1, nb)

        @pl.loop(0, _CH // 16)
        def _scal(q):
            qo = q * 16
            a = plsc.load_gather(as_v, [srcb[b][pl.ds(qo, 16)]])
            d = plsc.load_gather(ad_v, [dstb[b][pl.ds(qo, 16)]])
            alpha = a + d + aeb[b][pl.ds(qo, 16)]
            exb[b][pl.ds(qo, 16)] = jnp.exp(_lrelu(alpha, 0.2))

        pltpu.make_async_copy(h_hbm.at[srcb[b]], rowsb[b], gsems[b]).wait()

        @pl.loop(0, _CH // 16)
        def _mul(q):
            exv = exb[b][pl.ds(q * 16, 16)]
            for jj in range(16):
                j = q * 16 + jj
                s = exv[jj]
                for c in range(8):
                    rowsb[b][j, pl.ds(c * 16, 16)] = (
                        rowsb[b][j, pl.ds(c * 16, 16)] * s)

        pltpu.async_copy(rowsb[b], numer_sp.at[dstb[b]], ssems[b], add=True)
        pltpu.async_copy(exb[b], denom_sp.at[dstb[b]], dsems[b], add=True)

        if not last:
            _wait_idx(g + 1, nb)
            pltpu.async_copy(h_hbm.at[srcb[nb]], rowsb[nb], gsems[nb])

    _issue_idx(0, 0)
    _issue_idx(1, 1)
    _wait_idx(0, 0)
    pltpu.async_copy(h_hbm.at[srcb[0]], rows0, gsems[0])

    @pl.loop(0, _NCH - 1, step=2)
    def _vec(g):
        _process(g, 0, 1, False)
        _process(g + 1, 1, 0, False)

    _process(_NCH - 1, 0, 1, True)
    _wait_scatters(0)

    plsc.subcore_barrier()

    @pl.when(sid < 15)
    def _():
        pltpu.sync_copy(numer_sp.at[pl.ds(row0, _ROW_CHUNK), :],
                        numer_out.at[cid, pl.ds(row0, _ROW_CHUNK), :])

    @pl.when(sid == 15)
    def _():
        pltpu.sync_copy(numer_sp.at[pl.ds(15 * _ROW_CHUNK, _LAST_ROWS), :],
                        numer_out.at[cid, pl.ds(15 * _ROW_CHUNK, _LAST_ROWS), :])

    @pl.when(sid == 0)
    def _():
        pltpu.sync_copy(denom_sp, denom_out.at[cid, 0])


def _sc_layer(src, dst, asv, adv, ae, h):
    mesh = plsc.VectorSubcoreMesh(core_axis_name="c", subcore_axis_name="s",
                                  num_cores=2, num_subcores=16)
    f = pl.kernel(
        _sc_layer_body,
        out_type=[
            jax.ShapeDtypeStruct((2, _N, _H), _f32),
            jax.ShapeDtypeStruct((2, 1, _N), _f32),
        ],
        mesh=mesh,
        compiler_params=pltpu.CompilerParams(needs_layout_passes=False),
        scratch_types=(
            [
                pltpu.VMEM((_N,), _f32),
                pltpu.VMEM((_N,), _f32),
            ]
            + [pltpu.VMEM((_CH,), _i32)] * 4
            + [pltpu.VMEM((_CH,), _f32)] * 4
            + [pltpu.VMEM((_CH, _H), _f32)] * 2
            + [pltpu.VMEM((16, _H), _f32)]
            + [pltpu.SemaphoreType.DMA] * 8
            + [
                pltpu.VMEM_SHARED((_N, _H), _f32),
                pltpu.VMEM_SHARED((_N,), _f32),
            ]
        ),
    )
    return f(src, dst, asv, adv, ae, h)


def _tc_mid_body(n0, n1, d0, d1, h1, as1, ad1, m1, b1, w2, s2, t2,
                 h2_ref, as2_ref, ad2_ref):
    exl = jnp.exp(_lrelu(as1[...] + ad1[...] + m1[...], 0.2))
    num = n0[...] + n1[...] + exl * h1[...]
    den = d0[...] + d1[...] + exl + 1e-16
    out = num / den + b1[...]
    hr = jnp.maximum(out, 0.0)
    h2 = jnp.dot(hr, w2[...], preferred_element_type=_f32)
    h2_ref[...] = h2
    as2_ref[...] = jnp.dot(h2, s2[...], preferred_element_type=_f32)
    ad2_ref[...] = jnp.dot(h2, t2[...], preferred_element_type=_f32)


def _tc_mid(n0, n1, d0, d1, h1, as1, ad1, m1, b1, w2, s2, t2):
    return pl.pallas_call(
        _tc_mid_body,
        out_shape=[
            jax.ShapeDtypeStruct((_N, _H), _f32),
            jax.ShapeDtypeStruct((_N, 1), _f32),
            jax.ShapeDtypeStruct((_N, 1), _f32),
        ],
    )(n0, n1, d0, d1, h1, as1, ad1, m1, b1, w2, s2, t2)


def _tc_head_body(n0, n1, d0, d1, h2, as2, ad2, m2, b2,
                  a1w, ab1, a2w, ab2, m1w, mb1, m2w, mb2, m3w, mb3,
                  ap_ref, ns_ref):
    exl = jnp.exp(_lrelu(as2[...] + ad2[...] + m2[...], 0.2))
    num = n0[...] + n1[...] + exl * h2[...]
    den = d0[...] + d1[...] + exl + 1e-16
    h = num / den + b2[...]

    emb = jnp.mean(h, axis=0, keepdims=True)
    a = _lrelu(jnp.dot(emb, a1w[...], preferred_element_type=_f32) + ab1[...], 0.01)
    a = _lrelu(jnp.dot(a, a2w[...], preferred_element_type=_f32) + ab2[...], 0.01)
    amx = jnp.max(a)
    e = jnp.exp(a - amx)
    ap_ref[...] = e / jnp.sum(e)

    z = _lrelu(jnp.dot(h, m1w[...], preferred_element_type=_f32) + mb1[...], 0.01)
    z = _lrelu(jnp.dot(z, m2w[...], preferred_element_type=_f32) + mb2[...], 0.01)
    t = jnp.dot(z, m3w[...], preferred_element_type=_f32)[:, 0:1] + mb3[...]
    ns_ref[...] = 1.0 / (1.0 + jnp.exp(-t))


def _tc_head(n0, n1, d0, d1, h2, as2, ad2, m2, b2,
             a1w, ab1, a2w, ab2, m1w, mb1, m2w, mb2, m3w, mb3):
    return pl.pallas_call(
        _tc_head_body,
        out_shape=[
            jax.ShapeDtypeStruct((1, _H), _f32),
            jax.ShapeDtypeStruct((_N, 1), _f32),
        ],
    )(n0, n1, d0, d1, h2, as2, ad2, m2, b2,
      a1w, ab1, a2w, ab2, m1w, mb1, m2w, mb2, m3w, mb3)


def kernel(x, edge_index, edge_attr, W1, att_src1, att_dst1, We1, att_e1, b1,
           W2, att_src2, att_dst2, We2, att_e2, b2, A1, ab1, A2, ab2,
           M1, mb1, M2, mb2, M3, mb3):
    asrc1 = att_src1[:, None]
    adst1 = att_dst1[:, None]
    asrc2 = att_src2[:, None]
    adst2 = att_dst2[:, None]
    ate1 = att_e1[:, None]
    ate2 = att_e2[:, None]
    b1r = b1[None, :]
    b2r = b2[None, :]
    ab1r = ab1[None, :]
    mb1r = mb1[None, :]
    mb2r = mb2[None, :]
    mb3r = mb3[None, :]
    a2p = jnp.zeros((_H, _H), _f32).at[:, :3].set(A2)
    ab2p = jnp.full((1, _H), -1e30, _f32).at[0, :3].set(ab2)
    m3p = jnp.zeros((_H, _H), _f32).at[:, 0:1].set(M3)

    v1 = We1 @ att_e1
    v2 = We2 @ att_e2
    rows_idx = (_EDIM * jnp.arange(_H)[:, None]
                + jnp.arange(_EDIM)[None, :]).reshape(-1)
    cols_idx = jnp.repeat(jnp.arange(_H), _EDIM)
    b1f = jnp.zeros((_H * _EDIM, _H), _f32).at[rows_idx, cols_idx].set(
        jnp.tile(v1, _H))
    b2f = jnp.zeros((_H * _EDIM, _H), _f32).at[rows_idx, cols_idx].set(
        jnp.tile(v2, _H))
    x_fold = edge_attr.reshape(_E // _H, _H * _EDIM)

    h1, as1, ad1 = _tc_prep(x, W1, asrc1, adst1)
    ae1, ae2, m1, m2 = _tc_edge(x_fold, b1f, b2f)

    src = edge_index[0]
    dst = edge_index[1]
    numer1, denom1 = _sc_layer(src, dst, as1.reshape(-1), ad1.reshape(-1),
                               ae1.reshape(-1), h1)
    h2, as2, ad2 = _tc_mid(numer1[0], numer1[1],
                           denom1[0, 0][:, None], denom1[1, 0][:, None],
                           h1, as1, ad1, m1, b1r, W2, asrc2, adst2)

    numer2, denom2 = _sc_layer(src, dst, as2.reshape(-1), ad2.reshape(-1),
                               ae2.reshape(-1), h2)
    ap, ns = _tc_head(numer2[0], numer2[1],
                      denom2[0, 0][:, None], denom2[1, 0][:, None],
                      h2, as2, ad2, m2, b2r,
                      A1, ab1r, a2p, ab2p, M1, mb1r, M2, mb2r, m3p, mb3r)

    return (ap[0, :3], ns[:, 0])


# speedup vs baseline: 39.9893x; 1.2638x over previous
import functools

import jax
import jax.numpy as jnp
from jax import lax
from jax.experimental import pallas as pl
from jax.experimental.pallas import tpu as pltpu
from jax.experimental.pallas import tpu_sc as plsc

_N = 10000
_E = 320000
_H = 128
_EDIM = 16

_NTILES = 32
_EPT = _E // _NTILES
_CH = 80
_NCH = _EPT // _CH
_ROW_CHUNK = 624
_LAST_ROWS = _N - 15 * _ROW_CHUNK

_f32 = jnp.float32
_i32 = jnp.int32


def _lrelu(x, slope):
    return jnp.maximum(x, x * slope)


def _tc_prep_body(x_ref, w_ref, asrc_ref, adst_ref, h_ref, as_ref, ad_ref):
    h = jnp.dot(x_ref[...], w_ref[...], preferred_element_type=_f32)
    h_ref[...] = h
    as_ref[...] = jnp.dot(h, asrc_ref[...], preferred_element_type=_f32)
    ad_ref[...] = jnp.dot(h, adst_ref[...], preferred_element_type=_f32)


def _tc_prep(x, w, asrc, adst):
    return pl.pallas_call(
        _tc_prep_body,
        out_shape=[
            jax.ShapeDtypeStruct((_N, _H), _f32),
            jax.ShapeDtypeStruct((_N, 1), _f32),
            jax.ShapeDtypeStruct((_N, 1), _f32),
        ],
    )(x, w, asrc, adst)


def _tc_edge_body(x_ref, b1_ref, b2_ref, ae1_ref, ae2_ref, m1_ref, m2_ref):
    a1 = jnp.dot(x_ref[...], b1_ref[...], preferred_element_type=_f32)
    a2 = jnp.dot(x_ref[...], b2_ref[...], preferred_element_type=_f32)
    ae1_ref[...] = a1
    ae2_ref[...] = a2
    m1_ref[...] = jnp.sum(a1).reshape(1, 1) * (1.0 / _E)
    m2_ref[...] = jnp.sum(a2).reshape(1, 1) * (1.0 / _E)


def _tc_edge(x_fold, b1, b2):
    r = _E // _H
    return pl.pallas_call(
        _tc_edge_body,
        out_shape=[
            jax.ShapeDtypeStruct((r, _H), _f32),
            jax.ShapeDtypeStruct((r, _H), _f32),
            jax.ShapeDtypeStruct((1, 1), _f32),
            jax.ShapeDtypeStruct((1, 1), _f32),
        ],
    )(x_fold, b1, b2)


_RING = 4


def _sc_layer_body(src_hbm, dst_hbm, as_hbm, ad_hbm, ae_hbm, h_hbm,
                   numer_out, denom_out, *scr):
    srcb = scr[0:4]
    dstb = scr[4:8]
    aeb = scr[8:12]
    asg = scr[12:16]
    adg = scr[16:20]
    exb = scr[20:24]
    rowsb = scr[24:28]
    zrows = scr[28]
    isems = scr[29:33]
    scsems = scr[33:37]
    gsems = scr[37:41]
    ssems = scr[41:45]
    dsems = scr[45:49]
    numer_sp = scr[49]
    denom_sp = scr[50]

    cid = lax.axis_index("c")
    sid = lax.axis_index("s")
    wid = cid * 16 + sid
    base = wid * _EPT

    @pl.loop(0, 16)
    def _zz(j):
        for c in range(8):
            zrows[j, pl.ds(c * 16, 16)] = jnp.zeros((16,), _f32)

    row0 = sid * _ROW_CHUNK
    for r in range(0, _LAST_ROWS, 16):
        pltpu.sync_copy(zrows.at[pl.ds(0, 16), :],
                        numer_sp.at[pl.ds(row0 + r, 16), :])

    @pl.when(sid == 0)
    def _():
        for r in range(0, _N, 128):
            n = min(128, _N - r)
            pltpu.sync_copy(zrows.at[0, pl.ds(0, n)],
                            denom_sp.at[pl.ds(r, n)])

    plsc.subcore_barrier()

    def _maybe(cond, fn):
        if isinstance(cond, bool):
            if cond:
                fn()
        else:
            pl.when(cond)(fn)

    def _issue_idx(cg, k):
        off = base + cg * _CH
        pltpu.async_copy(src_hbm.at[pl.ds(off, _CH)], srcb[k], isems[k])
        pltpu.async_copy(dst_hbm.at[pl.ds(off, _CH)], dstb[k], isems[k])
        pltpu.async_copy(ae_hbm.at[pl.ds(off, _CH)], aeb[k], isems[k])

    def _drain_idx(cg, k):
        off = base + cg * _CH
        pltpu.make_async_copy(src_hbm.at[pl.ds(off, _CH)], srcb[k],
                              isems[k]).wait()
        pltpu.make_async_copy(dst_hbm.at[pl.ds(off, _CH)], dstb[k],
                              isems[k]).wait()
        pltpu.make_async_copy(ae_hbm.at[pl.ds(off, _CH)], aeb[k],
                              isems[k]).wait()

    def _issue_gathers(k):
        pltpu.async_copy(as_hbm.at[srcb[k]], asg[k], scsems[k])
        pltpu.async_copy(ad_hbm.at[dstb[k]], adg[k], scsems[k])
        pltpu.async_copy(h_hbm.at[srcb[k]], rowsb[k], gsems[k])

    def _drain_gathers(k):
        pltpu.make_async_copy(as_hbm.at[srcb[k]], asg[k], scsems[k]).wait()
        pltpu.make_async_copy(ad_hbm.at[dstb[k]], adg[k], scsems[k]).wait()
        pltpu.make_async_copy(h_hbm.at[srcb[k]], rowsb[k], gsems[k]).wait()

    def _issue_scatters(k):
        pltpu.async_copy(rowsb[k], numer_sp.at[dstb[k]], ssems[k], add=True)
        pltpu.async_copy(exb[k], denom_sp.at[dstb[k]], dsems[k], add=True)

    def _drain_scatters(k):
        pltpu.make_async_copy(rowsb[k], numer_sp.at[dstb[k]], ssems[k]).wait()
        pltpu.make_async_copy(exb[k], denom_sp.at[dstb[k]], dsems[k]).wait()

    def _process(cg, k):
        k2 = (k + 2) % _RING
        k1 = (k + 1) % _RING
        _maybe(cg >= 2 if isinstance(cg, int) else cg >= 2,
               lambda: _drain_scatters(k2))
        _maybe(cg <= _NCH - 3, lambda: _issue_idx(cg + 2, k2))
        def _g1():
            _drain_idx(cg + 1, k1)
            _issue_gathers(k1)
        _maybe(cg <= _NCH - 2, _g1)
        _drain_gathers(k)

        @pl.loop(0, _CH // 16)
        def _scal(q):
            qo = q * 16
            alpha = (asg[k][pl.ds(qo, 16)] + adg[k][pl.ds(qo, 16)]
                     + aeb[k][pl.ds(qo, 16)])
            exb[k][pl.ds(qo, 16)] = jnp.exp(_lrelu(alpha, 0.2))

        @pl.loop(0, _CH // 16)
        def _mul(q):
            exv = exb[k][pl.ds(q * 16, 16)]
            for jj in range(16):
                j = q * 16 + jj
                s = exv[jj]
                for c in range(8):
                    rowsb[k][j, pl.ds(c * 16, 16)] = (
                        rowsb[k][j, pl.ds(c * 16, 16)] * s)

        _issue_scatters(k)

    _issue_idx(0, 0)
    _issue_idx(1, 1)
    _drain_idx(0, 0)
    _issue_gathers(0)

    @pl.loop(0, _NCH - 1, step=_RING)
    def _vec(g):
        for o in range(_RING):
            _process(g + o, o)

    _process(_NCH - 1, 0)
    _drain_scatters(3)
    _drain_scatters(0)

    plsc.subcore_barrier()

    @pl.when(sid < 15)
    def _():
        pltpu.sync_copy(numer_sp.at[pl.ds(row0, _ROW_CHUNK), :],
                        numer_out.at[cid, pl.ds(row0, _ROW_CHUNK), :])

    @pl.when(sid == 15)
    def _():
        pltpu.sync_copy(numer_sp.at[pl.ds(15 * _ROW_CHUNK, _LAST_ROWS), :],
                        numer_out.at[cid, pl.ds(15 * _ROW_CHUNK, _LAST_ROWS), :])

    @pl.when(sid == 0)
    def _():
        pltpu.sync_copy(denom_sp, denom_out.at[cid, 0])


def _sc_layer(src, dst, asv, adv, ae, h):
    mesh = plsc.VectorSubcoreMesh(core_axis_name="c", subcore_axis_name="s",
                                  num_cores=2, num_subcores=16)
    f = pl.kernel(
        _sc_layer_body,
        out_type=[
            jax.ShapeDtypeStruct((2, _N, _H), _f32),
            jax.ShapeDtypeStruct((2, 1, _N), _f32),
        ],
        mesh=mesh,
        compiler_params=pltpu.CompilerParams(needs_layout_passes=False),
        scratch_types=(
            [pltpu.VMEM((_CH,), _i32)] * 8
            + [pltpu.VMEM((_CH,), _f32)] * 16
            + [pltpu.VMEM((_CH, _H), _f32)] * 4
            + [pltpu.VMEM((16, _H), _f32)]
            + [pltpu.SemaphoreType.DMA] * 20
            + [
                pltpu.VMEM_SHARED((_N, _H), _f32),
                pltpu.VMEM_SHARED((_N,), _f32),
            ]
        ),
    )
    return f(src, dst, asv, adv, ae, h)


def _tc_mid_body(n0, n1, d0, d1, h1, as1, ad1, m1, b1, w2, s2, t2,
                 h2_ref, as2_ref, ad2_ref):
    exl = jnp.exp(_lrelu(as1[...] + ad1[...] + m1[...], 0.2))
    num = n0[...] + n1[...] + exl * h1[...]
    den = d0[...] + d1[...] + exl + 1e-16
    out = num / den + b1[...]
    hr = jnp.maximum(out, 0.0)
    h2 = jnp.dot(hr, w2[...], preferred_element_type=_f32)
    h2_ref[...] = h2
    as2_ref[...] = jnp.dot(h2, s2[...], preferred_element_type=_f32)
    ad2_ref[...] = jnp.dot(h2, t2[...], preferred_element_type=_f32)


def _tc_mid(n0, n1, d0, d1, h1, as1, ad1, m1, b1, w2, s2, t2):
    return pl.pallas_call(
        _tc_mid_body,
        out_shape=[
            jax.ShapeDtypeStruct((_N, _H), _f32),
            jax.ShapeDtypeStruct((_N, 1), _f32),
            jax.ShapeDtypeStruct((_N, 1), _f32),
        ],
    )(n0, n1, d0, d1, h1, as1, ad1, m1, b1, w2, s2, t2)


def _tc_head_body(n0, n1, d0, d1, h2, as2, ad2, m2, b2,
                  a1w, ab1, a2w, ab2, m1w, mb1, m2w, mb2, m3w, mb3,
                  ap_ref, ns_ref):
    exl = jnp.exp(_lrelu(as2[...] + ad2[...] + m2[...], 0.2))
    num = n0[...] + n1[...] + exl * h2[...]
    den = d0[...] + d1[...] + exl + 1e-16
    h = num / den + b2[...]

    emb = jnp.mean(h, axis=0, keepdims=True)
    a = _lrelu(jnp.dot(emb, a1w[...], preferred_element_type=_f32) + ab1[...], 0.01)
    a = _lrelu(jnp.dot(a, a2w[...], preferred_element_type=_f32) + ab2[...], 0.01)
    amx = jnp.max(a)
    e = jnp.exp(a - amx)
    ap_ref[...] = e / jnp.sum(e)

    z = _lrelu(jnp.dot(h, m1w[...], preferred_element_type=_f32) + mb1[...], 0.01)
    z = _lrelu(jnp.dot(z, m2w[...], preferred_element_type=_f32) + mb2[...], 0.01)
    t = jnp.dot(z, m3w[...], preferred_element_type=_f32)[:, 0:1] + mb3[...]
    ns_ref[...] = 1.0 / (1.0 + jnp.exp(-t))


def _tc_head(n0, n1, d0, d1, h2, as2, ad2, m2, b2,
             a1w, ab1, a2w, ab2, m1w, mb1, m2w, mb2, m3w, mb3):
    return pl.pallas_call(
        _tc_head_body,
        out_shape=[
            jax.ShapeDtypeStruct((1, _H), _f32),
            jax.ShapeDtypeStruct((_N, 1), _f32),
        ],
    )(n0, n1, d0, d1, h2, as2, ad2, m2, b2,
      a1w, ab1, a2w, ab2, m1w, mb1, m2w, mb2, m3w, mb3)


def kernel(x, edge_index, edge_attr, W1, att_src1, att_dst1, We1, att_e1, b1,
           W2, att_src2, att_dst2, We2, att_e2, b2, A1, ab1, A2, ab2,
           M1, mb1, M2, mb2, M3, mb3):
    asrc1 = att_src1[:, None]
    adst1 = att_dst1[:, None]
    asrc2 = att_src2[:, None]
    adst2 = att_dst2[:, None]
    ate1 = att_e1[:, None]
    ate2 = att_e2[:, None]
    b1r = b1[None, :]
    b2r = b2[None, :]
    ab1r = ab1[None, :]
    mb1r = mb1[None, :]
    mb2r = mb2[None, :]
    mb3r = mb3[None, :]
    a2p = jnp.zeros((_H, _H), _f32).at[:, :3].set(A2)
    ab2p = jnp.full((1, _H), -1e30, _f32).at[0, :3].set(ab2)
    m3p = jnp.zeros((_H, _H), _f32).at[:, 0:1].set(M3)

    v1 = We1 @ att_e1
    v2 = We2 @ att_e2
    rows_idx = (_EDIM * jnp.arange(_H)[:, None]
                + jnp.arange(_EDIM)[None, :]).reshape(-1)
    cols_idx = jnp.repeat(jnp.arange(_H), _EDIM)
    b1f = jnp.zeros((_H * _EDIM, _H), _f32).at[rows_idx, cols_idx].set(
        jnp.tile(v1, _H))
    b2f = jnp.zeros((_H * _EDIM, _H), _f32).at[rows_idx, cols_idx].set(
        jnp.tile(v2, _H))
    x_fold = edge_attr.reshape(_E // _H, _H * _EDIM)

    h1, as1, ad1 = _tc_prep(x, W1, asrc1, adst1)
    ae1, ae2, m1, m2 = _tc_edge(x_fold, b1f, b2f)

    src = edge_index[0]
    dst = edge_index[1]
    numer1, denom1 = _sc_layer(src, dst, as1.reshape(-1), ad1.reshape(-1),
                               ae1.reshape(-1), h1)
    h2, as2, ad2 = _tc_mid(numer1[0], numer1[1],
                           denom1[0, 0][:, None], denom1[1, 0][:, None],
                           h1, as1, ad1, m1, b1r, W2, asrc2, adst2)

    numer2, denom2 = _sc_layer(src, dst, as2.reshape(-1), ad2.reshape(-1),
                               ae2.reshape(-1), h2)
    ap, ns = _tc_head(numer2[0], numer2[1],
                      denom2[0, 0][:, None], denom2[1, 0][:, None],
                      h2, as2, ad2, m2, b2r,
                      A1, ab1r, a2p, ab2p, M1, mb1r, M2, mb2r, m3p, mb3r)

    return (ap[0, :3], ns[:, 0])


# speedup vs baseline: 40.8667x; 1.0219x over previous
import functools

import jax
import jax.numpy as jnp
from jax import lax
from jax.experimental import pallas as pl
from jax.experimental.pallas import tpu as pltpu
from jax.experimental.pallas import tpu_sc as plsc

_N = 10000
_E = 320000
_H = 128
_EDIM = 16

_NTILES = 32
_EPT = _E // _NTILES
_CH = 80
_NCH = _EPT // _CH
_ROW_CHUNK = 624
_LAST_ROWS = _N - 15 * _ROW_CHUNK

_f32 = jnp.float32
_i32 = jnp.int32


def _lrelu(x, slope):
    return jnp.maximum(x, x * slope)


def _tc_prep_body(x_ref, w_ref, asrc_ref, adst_ref, xf_ref, b1_ref, b2_ref,
                  h_ref, as_ref, ad_ref, ae1_ref, ae2_ref, m1_ref, m2_ref):
    h = jnp.dot(x_ref[...], w_ref[...], preferred_element_type=_f32)
    h_ref[...] = h
    as_ref[...] = jnp.dot(h, asrc_ref[...], preferred_element_type=_f32)
    ad_ref[...] = jnp.dot(h, adst_ref[...], preferred_element_type=_f32)
    a1 = jnp.dot(xf_ref[...], b1_ref[...], preferred_element_type=_f32)
    a2 = jnp.dot(xf_ref[...], b2_ref[...], preferred_element_type=_f32)
    ae1_ref[...] = a1
    ae2_ref[...] = a2
    m1_ref[...] = jnp.sum(a1).reshape(1, 1) * (1.0 / _E)
    m2_ref[...] = jnp.sum(a2).reshape(1, 1) * (1.0 / _E)


def _tc_prep(x, w, asrc, adst, x_fold, b1, b2):
    return pl.pallas_call(
        _tc_prep_body,
        out_shape=[
            jax.ShapeDtypeStruct((_N, _H), _f32),
            jax.ShapeDtypeStruct((_N, 1), _f32),
            jax.ShapeDtypeStruct((_N, 1), _f32),
            jax.ShapeDtypeStruct((_E // _H, _H), _f32),
            jax.ShapeDtypeStruct((_E // _H, _H), _f32),
            jax.ShapeDtypeStruct((1, 1), _f32),
            jax.ShapeDtypeStruct((1, 1), _f32),
        ],
    )(x, w, asrc, adst, x_fold, b1, b2)


_RING = 4


def _sc_layer_body(src_hbm, dst_hbm, as_hbm, ad_hbm, ae_hbm, h_hbm,
                   numer_out, denom_out, *scr):
    srcb = scr[0:4]
    dstb = scr[4:8]
    aeb = scr[8:12]
    asg = scr[12:16]
    adg = scr[16:20]
    exb = scr[20:24]
    rowsb = scr[24:28]
    zrows = scr[28]
    isems = scr[29:33]
    scsems = scr[33:37]
    gsems = scr[37:41]
    ssems = scr[41:45]
    dsems = scr[45:49]
    numer_sp = scr[49]
    denom_sp = scr[50]

    cid = lax.axis_index("c")
    sid = lax.axis_index("s")
    wid = cid * 16 + sid
    base = wid * _EPT

    @pl.loop(0, 16)
    def _zz(j):
        for c in range(8):
            zrows[j, pl.ds(c * 16, 16)] = jnp.zeros((16,), _f32)

    row0 = sid * _ROW_CHUNK
    for r in range(0, _LAST_ROWS, 16):
        pltpu.sync_copy(zrows.at[pl.ds(0, 16), :],
                        numer_sp.at[pl.ds(row0 + r, 16), :])

    @pl.when(sid == 0)
    def _():
        for r in range(0, _N, 128):
            n = min(128, _N - r)
            pltpu.sync_copy(zrows.at[0, pl.ds(0, n)],
                            denom_sp.at[pl.ds(r, n)])

    plsc.subcore_barrier()

    def _maybe(cond, fn):
        if isinstance(cond, bool):
            if cond:
                fn()
        else:
            pl.when(cond)(fn)

    def _issue_idx(cg, k):
        off = base + cg * _CH
        pltpu.async_copy(src_hbm.at[pl.ds(off, _CH)], srcb[k], isems[k])
        pltpu.async_copy(dst_hbm.at[pl.ds(off, _CH)], dstb[k], isems[k])
        pltpu.async_copy(ae_hbm.at[pl.ds(off, _CH)], aeb[k], isems[k])

    def _drain_idx(cg, k):
        off = base + cg * _CH
        pltpu.make_async_copy(src_hbm.at[pl.ds(off, _CH)], srcb[k],
                              isems[k]).wait()
        pltpu.make_async_copy(dst_hbm.at[pl.ds(off, _CH)], dstb[k],
                              isems[k]).wait()
        pltpu.make_async_copy(ae_hbm.at[pl.ds(off, _CH)], aeb[k],
                              isems[k]).wait()

    def _issue_gathers(k):
        pltpu.async_copy(as_hbm.at[srcb[k]], asg[k], scsems[k])
        pltpu.async_copy(ad_hbm.at[dstb[k]], adg[k], scsems[k])
        pltpu.async_copy(h_hbm.at[srcb[k]], rowsb[k], gsems[k])

    def _drain_gathers(k):
        pltpu.make_async_copy(as_hbm.at[srcb[k]], asg[k], scsems[k]).wait()
        pltpu.make_async_copy(ad_hbm.at[dstb[k]], adg[k], scsems[k]).wait()
        pltpu.make_async_copy(h_hbm.at[srcb[k]], rowsb[k], gsems[k]).wait()

    def _issue_scatters(k):
        pltpu.async_copy(rowsb[k], numer_sp.at[dstb[k]], ssems[k], add=True)
        pltpu.async_copy(exb[k], denom_sp.at[dstb[k]], dsems[k], add=True)

    def _drain_scatters(k):
        pltpu.make_async_copy(rowsb[k], numer_sp.at[dstb[k]], ssems[k]).wait()
        pltpu.make_async_copy(exb[k], denom_sp.at[dstb[k]], dsems[k]).wait()

    def _process(cg, k):
        k2 = (k + 2) % _RING
        k1 = (k + 1) % _RING
        _maybe(cg >= 2 if isinstance(cg, int) else cg >= 2,
               lambda: _drain_scatters(k2))
        _maybe(cg <= _NCH - 3, lambda: _issue_idx(cg + 2, k2))
        def _g1():
            _drain_idx(cg + 1, k1)
            _issue_gathers(k1)
        _maybe(cg <= _NCH - 2, _g1)
        _drain_gathers(k)

        @pl.loop(0, _CH // 16)
        def _scal(q):
            qo = q * 16
            alpha = (asg[k][pl.ds(qo, 16)] + adg[k][pl.ds(qo, 16)]
                     + aeb[k][pl.ds(qo, 16)])
            exv = jnp.exp(_lrelu(alpha, 0.2))
            exb[k][pl.ds(qo, 16)] = exv
            for jj in range(16):
                j = qo + jj
                s = exv[jj]
                for c in range(8):
                    rowsb[k][j, pl.ds(c * 16, 16)] = (
                        rowsb[k][j, pl.ds(c * 16, 16)] * s)

        _issue_scatters(k)

    _issue_idx(0, 0)
    _issue_idx(1, 1)
    _drain_idx(0, 0)
    _issue_gathers(0)

    @pl.loop(0, _NCH - 1, step=_RING)
    def _vec(g):
        for o in range(_RING):
            _process(g + o, o)

    _process(_NCH - 1, 0)
    _drain_scatters(3)
    _drain_scatters(0)

    plsc.subcore_barrier()

    @pl.when(sid < 15)
    def _():
        pltpu.sync_copy(numer_sp.at[pl.ds(row0, _ROW_CHUNK), :],
                        numer_out.at[cid, pl.ds(row0, _ROW_CHUNK), :])

    @pl.when(sid == 15)
    def _():
        pltpu.sync_copy(numer_sp.at[pl.ds(15 * _ROW_CHUNK, _LAST_ROWS), :],
                        numer_out.at[cid, pl.ds(15 * _ROW_CHUNK, _LAST_ROWS), :])

    @pl.when(sid == 0)
    def _():
        pltpu.sync_copy(denom_sp, denom_out.at[cid, 0])


def _sc_layer(src, dst, asv, adv, ae, h):
    mesh = plsc.VectorSubcoreMesh(core_axis_name="c", subcore_axis_name="s",
                                  num_cores=2, num_subcores=16)
    f = pl.kernel(
        _sc_layer_body,
        out_type=[
            jax.ShapeDtypeStruct((2, _N, _H), _f32),
            jax.ShapeDtypeStruct((2, 1, _N), _f32),
        ],
        mesh=mesh,
        compiler_params=pltpu.CompilerParams(needs_layout_passes=False),
        scratch_types=(
            [pltpu.VMEM((_CH,), _i32)] * 8
            + [pltpu.VMEM((_CH,), _f32)] * 16
            + [pltpu.VMEM((_CH, _H), _f32)] * 4
            + [pltpu.VMEM((16, _H), _f32)]
            + [pltpu.SemaphoreType.DMA] * 20
            + [
                pltpu.VMEM_SHARED((_N, _H), _f32),
                pltpu.VMEM_SHARED((_N,), _f32),
            ]
        ),
    )
    return f(src, dst, asv, adv, ae, h)


def _tc_mid_body(n0, n1, d0, d1, h1, as1, ad1, m1, b1, w2, s2, t2,
                 h2_ref, as2_ref, ad2_ref):
    exl = jnp.exp(_lrelu(as1[...] + ad1[...] + m1[...], 0.2))
    num = n0[...] + n1[...] + exl * h1[...]
    den = d0[...] + d1[...] + exl + 1e-16
    out = num / den + b1[...]
    hr = jnp.maximum(out, 0.0)
    h2 = jnp.dot(hr, w2[...], preferred_element_type=_f32)
    h2_ref[...] = h2
    as2_ref[...] = jnp.dot(h2, s2[...], preferred_element_type=_f32)
    ad2_ref[...] = jnp.dot(h2, t2[...], preferred_element_type=_f32)


def _tc_mid(n0, n1, d0, d1, h1, as1, ad1, m1, b1, w2, s2, t2):
    return pl.pallas_call(
        _tc_mid_body,
        out_shape=[
            jax.ShapeDtypeStruct((_N, _H), _f32),
            jax.ShapeDtypeStruct((_N, 1), _f32),
            jax.ShapeDtypeStruct((_N, 1), _f32),
        ],
    )(n0, n1, d0, d1, h1, as1, ad1, m1, b1, w2, s2, t2)


def _tc_head_body(n0, n1, d0, d1, h2, as2, ad2, m2, b2,
                  a1w, ab1, a2w, ab2, m1w, mb1, m2w, mb2, m3w, mb3,
                  ap_ref, ns_ref):
    exl = jnp.exp(_lrelu(as2[...] + ad2[...] + m2[...], 0.2))
    num = n0[...] + n1[...] + exl * h2[...]
    den = d0[...] + d1[...] + exl + 1e-16
    h = num / den + b2[...]

    emb = jnp.mean(h, axis=0, keepdims=True)
    a = _lrelu(jnp.dot(emb, a1w[...], preferred_element_type=_f32) + ab1[...], 0.01)
    a = _lrelu(jnp.dot(a, a2w[...], preferred_element_type=_f32) + ab2[...], 0.01)
    amx = jnp.max(a)
    e = jnp.exp(a - amx)
    ap_ref[...] = e / jnp.sum(e)

    z = _lrelu(jnp.dot(h, m1w[...], preferred_element_type=_f32) + mb1[...], 0.01)
    z = _lrelu(jnp.dot(z, m2w[...], preferred_element_type=_f32) + mb2[...], 0.01)
    t = jnp.dot(z, m3w[...], preferred_element_type=_f32)[:, 0:1] + mb3[...]
    ns_ref[...] = 1.0 / (1.0 + jnp.exp(-t))


def _tc_head(n0, n1, d0, d1, h2, as2, ad2, m2, b2,
             a1w, ab1, a2w, ab2, m1w, mb1, m2w, mb2, m3w, mb3):
    return pl.pallas_call(
        _tc_head_body,
        out_shape=[
            jax.ShapeDtypeStruct((1, _H), _f32),
            jax.ShapeDtypeStruct((_N, 1), _f32),
        ],
    )(n0, n1, d0, d1, h2, as2, ad2, m2, b2,
      a1w, ab1, a2w, ab2, m1w, mb1, m2w, mb2, m3w, mb3)


def kernel(x, edge_index, edge_attr, W1, att_src1, att_dst1, We1, att_e1, b1,
           W2, att_src2, att_dst2, We2, att_e2, b2, A1, ab1, A2, ab2,
           M1, mb1, M2, mb2, M3, mb3):
    asrc1 = att_src1[:, None]
    adst1 = att_dst1[:, None]
    asrc2 = att_src2[:, None]
    adst2 = att_dst2[:, None]
    ate1 = att_e1[:, None]
    ate2 = att_e2[:, None]
    b1r = b1[None, :]
    b2r = b2[None, :]
    ab1r = ab1[None, :]
    mb1r = mb1[None, :]
    mb2r = mb2[None, :]
    mb3r = mb3[None, :]
    a2p = jnp.zeros((_H, _H), _f32).at[:, :3].set(A2)
    ab2p = jnp.full((1, _H), -1e30, _f32).at[0, :3].set(ab2)
    m3p = jnp.zeros((_H, _H), _f32).at[:, 0:1].set(M3)

    v1 = We1 @ att_e1
    v2 = We2 @ att_e2
    rows_idx = (_EDIM * jnp.arange(_H)[:, None]
                + jnp.arange(_EDIM)[None, :]).reshape(-1)
    cols_idx = jnp.repeat(jnp.arange(_H), _EDIM)
    b1f = jnp.zeros((_H * _EDIM, _H), _f32).at[rows_idx, cols_idx].set(
        jnp.tile(v1, _H))
    b2f = jnp.zeros((_H * _EDIM, _H), _f32).at[rows_idx, cols_idx].set(
        jnp.tile(v2, _H))
    x_fold = edge_attr.reshape(_E // _H, _H * _EDIM)

    h1, as1, ad1, ae1, ae2, m1, m2 = _tc_prep(x, W1, asrc1, adst1,
                                              x_fold, b1f, b2f)

    src = edge_index[0]
    dst = edge_index[1]
    numer1, denom1 = _sc_layer(src, dst, as1.reshape(-1), ad1.reshape(-1),
                               ae1.reshape(-1), h1)
    h2, as2, ad2 = _tc_mid(numer1[0], numer1[1],
                           denom1[0, 0][:, None], denom1[1, 0][:, None],
                           h1, as1, ad1, m1, b1r, W2, asrc2, adst2)

    numer2, denom2 = _sc_layer(src, dst, as2.reshape(-1), ad2.reshape(-1),
                               ae2.reshape(-1), h2)
    ap, ns = _tc_head(numer2[0], numer2[1],
                      denom2[0, 0][:, None], denom2[1, 0][:, None],
                      h2, as2, ad2, m2, b2r,
                      A1, ab1r, a2p, ab2p, M1, mb1r, M2, mb2r, m3p, mb3r)

    return (ap[0, :3], ns[:, 0])
